# baseline (speedup 1.0000x reference)
import functools

import jax
import jax.numpy as jnp
from jax import lax
from jax.experimental import pallas as pl
from jax.experimental.pallas import tpu as pltpu
from jax.experimental.pallas import tpu_sc as plsc

_NUM_WORKERS = 32
_LIGHT = 28


def kernel(indices, embedding_weight):
    batch, tokens = indices.shape
    vocab, dim = embedding_weight.shape
    rows = batch * tokens
    assert _LIGHT * 24 + (_NUM_WORKERS - _LIGHT) * 32 == rows

    idx_flat = indices.reshape(-1).astype(jnp.int32)
    mesh = plsc.VectorSubcoreMesh(core_axis_name="c", subcore_axis_name="s")

    @functools.partial(
        pl.kernel,
        mesh=mesh,
        out_type=jax.ShapeDtypeStruct((rows, dim), jnp.float32),
        scratch_types=[
            pltpu.VMEM((32,), jnp.int32),
            pltpu.VMEM((8, dim), jnp.float32),
            pltpu.VMEM((8, dim), jnp.float32),
            pltpu.VMEM((8, dim), jnp.float32),
            pltpu.SemaphoreType.DMA,
            pltpu.SemaphoreType.DMA,
        ],
    )
    def gather_kernel(table_hbm, idx_hbm, out_hbm, idx_v, b0, b1, b2, gsem, wsem):
        wid = lax.axis_index("s") * 2 + lax.axis_index("c")
        heavy = wid >= _LIGHT
        off = jnp.where(heavy, _LIGHT * 24 + (wid - _LIGHT) * 32, wid * 24)
        bufs = (b0, b1, b2)

        pltpu.sync_copy(idx_hbm.at[pl.ds(off, 32)], idx_v)

        def gather(k, buf):
            return pltpu.async_copy(
                table_hbm.at[idx_v.at[pl.ds(k * 8, 8)]], buf, gsem
            )

        def write(k, buf):
            return pltpu.async_copy(
                buf, out_hbm.at[pl.ds(off + k * 8, 8)], wsem
            )

        g = [gather(k, bufs[k]) for k in range(3)]
        w = []
        for k in range(3):
            g[k].wait()
            w.append(write(k, bufs[k]))
        w[0].wait()

        @pl.when(heavy)
        def _():
            g3 = gather(3, b0)
            g3.wait()
            write(3, b0).wait()

        w[1].wait()
        w[2].wait()

    out = gather_kernel(embedding_weight, idx_flat)
    return out.reshape(batch, tokens, dim)
